# baseline (speedup 1.0000x reference)
import functools

import jax
import jax.numpy as jnp
from jax import lax
from jax.experimental import pallas as pl
from jax.experimental.pallas import tpu as pltpu
from jax.experimental.pallas import tpu_sc as plsc

_N = 10000
_E = 320000
_C = 128
_NT = 32
_EPT = _E // _NT
_K = 80
_NCH = _EPT // _K
_RPT = _N // 16
_BN = 2000


def _proj_body(x_ref, ws_ref, wd_ref, atts_ref, attd_ref,
               h_ref, as_ref, ad_ref, b_ref, ms_ref, md_ref):
    i = pl.program_id(0)
    ng = pl.num_programs(0)
    x = x_ref[...]
    h = jnp.dot(x, ws_ref[...], preferred_element_type=jnp.float32)
    h_ref[...] = h
    a_s = jnp.sum(h * atts_ref[...], axis=1, keepdims=True)
    as_ref[...] = a_s
    hd = jnp.dot(x, wd_ref[...], preferred_element_type=jnp.float32)
    a_d = jnp.sum(hd * attd_ref[...], axis=1, keepdims=True)
    ad_ref[...] = a_d

    @pl.when(i == 0)
    def _():
        ms_ref[0] = jnp.float32(-jnp.inf)
        md_ref[0] = jnp.float32(-jnp.inf)

    ms_ref[0] = jnp.maximum(ms_ref[0], jnp.max(a_s))
    md_ref[0] = jnp.maximum(md_ref[0], jnp.max(a_d))

    @pl.when(i == ng - 1)
    def _():
        b = jnp.maximum(ms_ref[0] + md_ref[0], 0.0)
        b_ref[...] = jnp.full((1, 16), b, jnp.float32)


def _proj(x, w_s, w_d, atts, attd):
    grid = _N // _BN
    return pl.pallas_call(
        _proj_body,
        grid=(grid,),
        in_specs=[
            pl.BlockSpec((_BN, _C), lambda i: (i, 0)),
            pl.BlockSpec((_C, _C), lambda i: (0, 0)),
            pl.BlockSpec((_C, _C), lambda i: (0, 0)),
            pl.BlockSpec((1, _C), lambda i: (0, 0)),
            pl.BlockSpec((1, _C), lambda i: (0, 0)),
        ],
        out_specs=[
            pl.BlockSpec((_BN, _C), lambda i: (i, 0)),
            pl.BlockSpec((_BN, 1), lambda i: (i, 0)),
            pl.BlockSpec((_BN, 1), lambda i: (i, 0)),
            pl.BlockSpec((1, 16), lambda i: (0, 0)),
        ],
        out_shape=[
            jax.ShapeDtypeStruct((_N, _C), jnp.float32),
            jax.ShapeDtypeStruct((_N, 1), jnp.float32),
            jax.ShapeDtypeStruct((_N, 1), jnp.float32),
            jax.ShapeDtypeStruct((1, 16), jnp.float32),
        ],
        scratch_shapes=[
            pltpu.SMEM((1,), jnp.float32),
            pltpu.SMEM((1,), jnp.float32),
        ],
    )(x, w_s, w_d, atts, attd)


def _gat_sc_body(h_hbm, si_hbm, di_hbm, as_hbm, ad_hbm, b_hbm,
                 acc_out, den_out,
                 sbuf, dbuf, gbuf, mbuf, avb, adb, bv,
                 acc_s, den_s, *sems):
    cid = lax.axis_index("c")
    sid = lax.axis_index("s")
    tg = cid * 16 + sid
    g0, g1 = gbuf
    m0, m1 = mbuf
    av0, av1 = avb
    ad0, ad1 = adb

    pltpu.sync_copy(si_hbm.at[pl.ds(pl.multiple_of(tg * _EPT, 8), _EPT)],
                    sbuf)
    pltpu.sync_copy(di_hbm.at[tg], dbuf)
    pltpu.sync_copy(b_hbm, bv)

    def _zrow(r, carry):
        for c8 in range(8):
            g0[r, pl.ds(c8 * 16, 16)] = jnp.zeros((16,), jnp.float32)
        return carry

    lax.fori_loop(0, _K, _zrow, 0)
    for g in range(_K // 16 + 1):
        m0[pl.ds(g * 16, 16)] = jnp.zeros((16,), jnp.float32)

    for k in range(8):
        start = pl.multiple_of(sid * 8 * _K + k * _K, _K)

        @pl.when(start < _N)
        def _():
            pltpu.sync_copy(g0, acc_s.at[pl.ds(start, _K)])
            pltpu.sync_copy(m0.at[pl.ds(16, _K)], den_s.at[pl.ds(start, _K)])

    plsc.subcore_barrier()

    gs0, gs1, as0, as1, ds0, ds1, rsem, msem = sems
    P0 = (g0, av0, ad0, m0, gs0, as0, ds0)
    P1 = (g1, av1, ad1, m1, gs1, as1, ds1)

    def srow(j1):
        return sbuf.at[pl.ds(pl.multiple_of(j1 * _K, 8), _K)]

    def fire_in(j1, P):
        gb, vb, db, gsem, asem, dsem = P[0], P[1], P[2], P[4], P[5], P[6]
        pltpu.async_copy(h_hbm.at[srow(j1)], gb, gsem)
        pltpu.async_copy(as_hbm.at[srow(j1)], vb, asem)
        pltpu.async_copy(ad_hbm.at[dbuf.at[j1]], db, dsem)

    fire_in(0, P0)

    def run_chunk(j, P, Q):
        gb, vb, db, mb, gsem, asem, dsem = P
        pltpu.make_async_copy(h_hbm.at[srow(j)], gb, gsem).wait()
        pltpu.make_async_copy(as_hbm.at[srow(j)], vb, asem).wait()
        pltpu.make_async_copy(ad_hbm.at[dbuf.at[j]], db, dsem).wait()

        @pl.when(j + 1 < _NCH)
        def _():
            fire_in(j + 1, Q)

        bb = bv[...]
        for g in range(_K // 16):
            e = vb[pl.ds(g * 16, 16)] + db[pl.ds(g * 16, 16)]
            e = jnp.where(e > 0, e, e * jnp.float32(0.2))
            mb[pl.ds(16 + g * 16, 16)] = jnp.exp(e - bb)
        for k in range(_K):
            mk = plsc.load_gather(mb, [jnp.full((16,), 16 + k, jnp.int32)])
            for r in range(8):
                gb[k, pl.ds(r * 16, 16)] = gb[k, pl.ds(r * 16, 16)] * mk
        rd = pltpu.async_copy(gb, acc_s.at[dbuf.at[j]], rsem, add=True)
        md = pltpu.async_copy(mb.at[pl.ds(16, _K)], den_s.at[dbuf.at[j]],
                              msem, add=True)
        rd.wait()
        md.wait()

    def _loop(j, carry):
        @pl.when(j % 2 == 0)
        def _():
            run_chunk(j, P0, P1)

        @pl.when(j % 2 == 1)
        def _():
            run_chunk(j, P1, P0)

        return carry

    lax.fori_loop(0, _NCH, _loop, 0)
    plsc.subcore_barrier()

    for k in range(8):
        start = pl.multiple_of(sid * 8 * _K + k * _K, _K)

        @pl.when(start < _N)
        def _():
            pltpu.sync_copy(acc_s.at[pl.ds(start, _K)],
                            acc_out.at[cid, pl.ds(start, _K)])

    @pl.when(sid == 0)
    def _():
        pltpu.sync_copy(den_s, den_out.at[cid, 0])


def _gat_sc(h, si, di, a_s, a_d, bvec):
    mesh = plsc.VectorSubcoreMesh(core_axis_name="c", subcore_axis_name="s")
    f = pl.kernel(
        _gat_sc_body,
        out_type=[
            jax.ShapeDtypeStruct((2, _N, _C), jnp.float32),
            jax.ShapeDtypeStruct((2, 1, _N), jnp.float32),
        ],
        mesh=mesh,
        compiler_params=pltpu.CompilerParams(needs_layout_passes=False),
        scratch_types=[
            pltpu.VMEM((_EPT,), jnp.int32),
            pltpu.VMEM((_NCH, _K), jnp.int32),
            (pltpu.VMEM((_K, _C), jnp.float32),
             pltpu.VMEM((_K, _C), jnp.float32)),
            (pltpu.VMEM((16 + _K,), jnp.float32),
             pltpu.VMEM((16 + _K,), jnp.float32)),
            (pltpu.VMEM((_K,), jnp.float32),
             pltpu.VMEM((_K,), jnp.float32)),
            (pltpu.VMEM((_K,), jnp.float32),
             pltpu.VMEM((_K,), jnp.float32)),
            pltpu.VMEM((16,), jnp.float32),
            pltpu.VMEM_SHARED((_N, _C), jnp.float32),
            pltpu.VMEM_SHARED((_N,), jnp.float32),
        ] + [pltpu.SemaphoreType.DMA] * 8,
    )
    return f(h, si, di, a_s, a_d, bvec)


def _epi_body(acc_ref, den_ref, bias_ref, o_ref):
    rows = acc_ref[0] + acc_ref[1]
    den = den_ref[0] + den_ref[1]
    o_ref[...] = jnp.tanh(rows / (den + jnp.float32(1e-16)) + bias_ref[...])


def _epi(acc, den, bias2):
    grid = _N // _BN
    return pl.pallas_call(
        _epi_body,
        grid=(grid,),
        in_specs=[
            pl.BlockSpec((2, _BN, _C), lambda i: (0, i, 0)),
            pl.BlockSpec((2, _BN, 1), lambda i: (0, i, 0)),
            pl.BlockSpec((1, _C), lambda i: (0, 0)),
        ],
        out_specs=pl.BlockSpec((_BN, _C), lambda i: (i, 0)),
        out_shape=jax.ShapeDtypeStruct((_N, _C), jnp.float32),
    )(acc, den, bias2)


def kernel(x, edge_index, W_src, W_dst, att_src, att_dst, bias):
    n, d = x.shape
    c = W_src.shape[1]
    h, a_s, a_d, bmax = _proj(x, W_src, W_dst,
                              att_src.reshape(1, c), att_dst.reshape(1, c))
    si = edge_index[0]
    di = edge_index[1].reshape(_NT, _NCH, _K)
    acc, den = _gat_sc(h, si, di, a_s.reshape(n), a_d.reshape(n),
                       bmax.reshape(16))
    return _epi(acc, den.reshape(2, n, 1), bias.reshape(1, c))


# speedup vs baseline: 1.5397x; 1.5397x over previous
import functools

import jax
import jax.numpy as jnp
from jax import lax
from jax.experimental import pallas as pl
from jax.experimental.pallas import tpu as pltpu
from jax.experimental.pallas import tpu_sc as plsc

_N = 10000
_E = 320000
_C = 128
_NT = 32
_EPT = _E // _NT
_K = 80
_NCH = _EPT // _K
_RPT = _N // 16
_BN = 2000


def _proj_body(x_ref, ws_ref, wd_ref, atts_ref, attd_ref,
               h_ref, as_ref, ad_ref, b_ref, ms_ref, md_ref):
    i = pl.program_id(0)
    ng = pl.num_programs(0)
    x = x_ref[...]
    h = jnp.dot(x, ws_ref[...], preferred_element_type=jnp.float32)
    h_ref[...] = h
    a_s = jnp.sum(h * atts_ref[...], axis=1, keepdims=True)
    as_ref[...] = a_s
    hd = jnp.dot(x, wd_ref[...], preferred_element_type=jnp.float32)
    a_d = jnp.sum(hd * attd_ref[...], axis=1, keepdims=True)
    ad_ref[...] = a_d

    @pl.when(i == 0)
    def _():
        ms_ref[0] = jnp.float32(-jnp.inf)
        md_ref[0] = jnp.float32(-jnp.inf)

    ms_ref[0] = jnp.maximum(ms_ref[0], jnp.max(a_s))
    md_ref[0] = jnp.maximum(md_ref[0], jnp.max(a_d))

    @pl.when(i == ng - 1)
    def _():
        b = jnp.maximum(ms_ref[0] + md_ref[0], 0.0)
        b_ref[...] = jnp.full((1, 16), b, jnp.float32)


def _proj(x, w_s, w_d, atts, attd):
    grid = _N // _BN
    return pl.pallas_call(
        _proj_body,
        grid=(grid,),
        in_specs=[
            pl.BlockSpec((_BN, _C), lambda i: (i, 0)),
            pl.BlockSpec((_C, _C), lambda i: (0, 0)),
            pl.BlockSpec((_C, _C), lambda i: (0, 0)),
            pl.BlockSpec((1, _C), lambda i: (0, 0)),
            pl.BlockSpec((1, _C), lambda i: (0, 0)),
        ],
        out_specs=[
            pl.BlockSpec((_BN, _C), lambda i: (i, 0)),
            pl.BlockSpec((_BN, 1), lambda i: (i, 0)),
            pl.BlockSpec((_BN, 1), lambda i: (i, 0)),
            pl.BlockSpec((1, 16), lambda i: (0, 0)),
        ],
        out_shape=[
            jax.ShapeDtypeStruct((_N, _C), jnp.float32),
            jax.ShapeDtypeStruct((_N, 1), jnp.float32),
            jax.ShapeDtypeStruct((_N, 1), jnp.float32),
            jax.ShapeDtypeStruct((1, 16), jnp.float32),
        ],
        scratch_shapes=[
            pltpu.SMEM((1,), jnp.float32),
            pltpu.SMEM((1,), jnp.float32),
        ],
    )(x, w_s, w_d, atts, attd)


def _gat_sc_body(h_hbm, si_hbm, di_hbm, as_hbm, ad_hbm, b_hbm,
                 acc_out, den_out,
                 sbuf, dbuf, gbuf, mbuf, avb, adb, bv,
                 acc_s, den_s, *sems):
    cid = lax.axis_index("c")
    sid = lax.axis_index("s")
    tg = cid * 16 + sid
    g0, g1 = gbuf
    m0, m1 = mbuf
    av0, av1 = avb
    ad0, ad1 = adb

    pltpu.sync_copy(si_hbm.at[pl.ds(pl.multiple_of(tg * _EPT, 8), _EPT)],
                    sbuf)
    pltpu.sync_copy(di_hbm.at[tg], dbuf)
    pltpu.sync_copy(b_hbm, bv)

    def _zrow(r, carry):
        for c8 in range(8):
            g0[r, pl.ds(c8 * 16, 16)] = jnp.zeros((16,), jnp.float32)
        return carry

    lax.fori_loop(0, _K, _zrow, 0)
    for g in range(_K // 16 + 1):
        m0[pl.ds(g * 16, 16)] = jnp.zeros((16,), jnp.float32)

    for k in range(8):
        start = pl.multiple_of(sid * 8 * _K + k * _K, _K)

        @pl.when(start < _N)
        def _():
            pltpu.sync_copy(g0, acc_s.at[pl.ds(start, _K)])
            pltpu.sync_copy(m0.at[pl.ds(16, _K)], den_s.at[pl.ds(start, _K)])

    plsc.subcore_barrier()

    gs0, gs1, as0, as1, ds0, ds1, rs0, rs1 = sems
    P0 = (g0, av0, ad0, m0, gs0, as0, ds0, rs0)
    P1 = (g1, av1, ad1, m1, gs1, as1, ds1, rs1)

    def srow(j1):
        return sbuf.at[pl.ds(pl.multiple_of(j1 * _K, 8), _K)]

    def fire_in(j1, P):
        gb, vb, db, gsem, asem, dsem = P[0], P[1], P[2], P[4], P[5], P[6]
        pltpu.async_copy(h_hbm.at[srow(j1)], gb, gsem)
        pltpu.async_copy(as_hbm.at[srow(j1)], vb, asem)
        pltpu.async_copy(ad_hbm.at[dbuf.at[j1]], db, dsem)

    fire_in(0, P0)

    def run_chunk(j, P, Q):
        gb, vb, db, mb, gsem, asem, dsem, rsem = P
        qgb, qrsem = Q[0], Q[7]
        pltpu.make_async_copy(h_hbm.at[srow(j)], gb, gsem).wait()
        pltpu.make_async_copy(as_hbm.at[srow(j)], vb, asem).wait()
        pltpu.make_async_copy(ad_hbm.at[dbuf.at[j]], db, dsem).wait()

        @pl.when(j + 1 < _NCH)
        def _():
            @pl.when(j >= 1)
            def _():
                pltpu.make_async_copy(qgb, acc_s.at[dbuf.at[j]],
                                      qrsem).wait()

            fire_in(j + 1, Q)

        bb = bv[...]
        for g in range(_K // 16):
            e = vb[pl.ds(g * 16, 16)] + db[pl.ds(g * 16, 16)]
            e = jnp.where(e > 0, e, e * jnp.float32(0.2))
            mb[pl.ds(16 + g * 16, 16)] = jnp.exp(e - bb)
        for k in range(_K):
            mk = plsc.load_gather(mb, [jnp.full((16,), 16 + k, jnp.int32)])
            for r in range(8):
                gb[k, pl.ds(r * 16, 16)] = gb[k, pl.ds(r * 16, 16)] * mk
        pltpu.async_copy(gb, acc_s.at[dbuf.at[j]], rsem, add=True)
        pltpu.sync_copy(mb.at[pl.ds(16, _K)], den_s.at[dbuf.at[j]], add=True)

    def _loop(j, carry):
        @pl.when(j % 2 == 0)
        def _():
            run_chunk(j, P0, P1)

        @pl.when(j % 2 == 1)
        def _():
            run_chunk(j, P1, P0)

        return carry

    lax.fori_loop(0, _NCH, _loop, 0)
    pltpu.make_async_copy(g0, acc_s.at[dbuf.at[0]], rs0).wait()
    pltpu.make_async_copy(g1, acc_s.at[dbuf.at[0]], rs1).wait()
    plsc.subcore_barrier()

    for k in range(8):
        start = pl.multiple_of(sid * 8 * _K + k * _K, _K)

        @pl.when(start < _N)
        def _():
            pltpu.sync_copy(acc_s.at[pl.ds(start, _K)],
                            acc_out.at[cid, pl.ds(start, _K)])

    @pl.when(sid == 0)
    def _():
        pltpu.sync_copy(den_s, den_out.at[cid, 0])


def _gat_sc(h, si, di, a_s, a_d, bvec):
    mesh = plsc.VectorSubcoreMesh(core_axis_name="c", subcore_axis_name="s")
    f = pl.kernel(
        _gat_sc_body,
        out_type=[
            jax.ShapeDtypeStruct((2, _N, _C), jnp.float32),
            jax.ShapeDtypeStruct((2, 1, _N), jnp.float32),
        ],
        mesh=mesh,
        compiler_params=pltpu.CompilerParams(needs_layout_passes=False),
        scratch_types=[
            pltpu.VMEM((_EPT,), jnp.int32),
            pltpu.VMEM((_NCH, _K), jnp.int32),
            (pltpu.VMEM((_K, _C), jnp.float32),
             pltpu.VMEM((_K, _C), jnp.float32)),
            (pltpu.VMEM((16 + _K,), jnp.float32),
             pltpu.VMEM((16 + _K,), jnp.float32)),
            (pltpu.VMEM((_K,), jnp.float32),
             pltpu.VMEM((_K,), jnp.float32)),
            (pltpu.VMEM((_K,), jnp.float32),
             pltpu.VMEM((_K,), jnp.float32)),
            pltpu.VMEM((16,), jnp.float32),
            pltpu.VMEM_SHARED((_N, _C), jnp.float32),
            pltpu.VMEM_SHARED((_N,), jnp.float32),
        ] + [pltpu.SemaphoreType.DMA] * 8,
    )
    return f(h, si, di, a_s, a_d, bvec)


def _epi_body(acc_ref, den_ref, bias_ref, o_ref):
    rows = acc_ref[0] + acc_ref[1]
    den = den_ref[0] + den_ref[1]
    o_ref[...] = jnp.tanh(rows / (den + jnp.float32(1e-16)) + bias_ref[...])


def _epi(acc, den, bias2):
    grid = _N // _BN
    return pl.pallas_call(
        _epi_body,
        grid=(grid,),
        in_specs=[
            pl.BlockSpec((2, _BN, _C), lambda i: (0, i, 0)),
            pl.BlockSpec((2, _BN, 1), lambda i: (0, i, 0)),
            pl.BlockSpec((1, _C), lambda i: (0, 0)),
        ],
        out_specs=pl.BlockSpec((_BN, _C), lambda i: (i, 0)),
        out_shape=jax.ShapeDtypeStruct((_N, _C), jnp.float32),
    )(acc, den, bias2)


def kernel(x, edge_index, W_src, W_dst, att_src, att_dst, bias):
    n, d = x.shape
    c = W_src.shape[1]
    h, a_s, a_d, bmax = _proj(x, W_src, W_dst,
                              att_src.reshape(1, c), att_dst.reshape(1, c))
    si = edge_index[0]
    di = edge_index[1].reshape(_NT, _NCH, _K)
    acc, den = _gat_sc(h, si, di, a_s.reshape(n), a_d.reshape(n),
                       bmax.reshape(16))
    return _epi(acc, den.reshape(2, n, 1), bias.reshape(1, c))


# speedup vs baseline: 1.5400x; 1.0002x over previous
import functools

import jax
import jax.numpy as jnp
from jax import lax
from jax.experimental import pallas as pl
from jax.experimental.pallas import tpu as pltpu
from jax.experimental.pallas import tpu_sc as plsc

_N = 10000
_E = 320000
_C = 128
_NT = 32
_EPT = _E // _NT
_K = 80
_NCH = _EPT // _K
_RPT = _N // 16
_BN = 2000


def _proj_body(x_ref, ws_ref, wd_ref, atts_ref, attd_ref,
               h_ref, as_ref, ad_ref, b_ref, ms_ref, md_ref):
    i = pl.program_id(0)
    ng = pl.num_programs(0)
    x = x_ref[...]
    h = jnp.dot(x, ws_ref[...], preferred_element_type=jnp.float32)
    h_ref[...] = h
    a_s = jnp.sum(h * atts_ref[...], axis=1, keepdims=True)
    as_ref[...] = a_s
    hd = jnp.dot(x, wd_ref[...], preferred_element_type=jnp.float32)
    a_d = jnp.sum(hd * attd_ref[...], axis=1, keepdims=True)
    ad_ref[...] = a_d

    @pl.when(i == 0)
    def _():
        ms_ref[0] = jnp.float32(-jnp.inf)
        md_ref[0] = jnp.float32(-jnp.inf)

    ms_ref[0] = jnp.maximum(ms_ref[0], jnp.max(a_s))
    md_ref[0] = jnp.maximum(md_ref[0], jnp.max(a_d))

    @pl.when(i == ng - 1)
    def _():
        b = jnp.maximum(ms_ref[0] + md_ref[0], 0.0)
        b_ref[...] = jnp.full((1, 16), b, jnp.float32)


def _proj(x, w_s, w_d, atts, attd):
    grid = _N // _BN
    return pl.pallas_call(
        _proj_body,
        grid=(grid,),
        in_specs=[
            pl.BlockSpec((_BN, _C), lambda i: (i, 0)),
            pl.BlockSpec((_C, _C), lambda i: (0, 0)),
            pl.BlockSpec((_C, _C), lambda i: (0, 0)),
            pl.BlockSpec((1, _C), lambda i: (0, 0)),
            pl.BlockSpec((1, _C), lambda i: (0, 0)),
        ],
        out_specs=[
            pl.BlockSpec((_BN, _C), lambda i: (i, 0)),
            pl.BlockSpec((_BN, 1), lambda i: (i, 0)),
            pl.BlockSpec((_BN, 1), lambda i: (i, 0)),
            pl.BlockSpec((1, 16), lambda i: (0, 0)),
        ],
        out_shape=[
            jax.ShapeDtypeStruct((_N, _C), jnp.float32),
            jax.ShapeDtypeStruct((_N, 1), jnp.float32),
            jax.ShapeDtypeStruct((_N, 1), jnp.float32),
            jax.ShapeDtypeStruct((1, 16), jnp.float32),
        ],
        scratch_shapes=[
            pltpu.SMEM((1,), jnp.float32),
            pltpu.SMEM((1,), jnp.float32),
        ],
    )(x, w_s, w_d, atts, attd)


def _gat_sc_body(h_hbm, si_hbm, di_hbm, as_hbm, ad_hbm, b_hbm,
                 acc_out, den_out,
                 sbuf, dbuf, gbuf, mbuf, avb, adb, bv,
                 acc_s, den_s, *sems):
    cid = lax.axis_index("c")
    sid = lax.axis_index("s")
    tg = cid * 16 + sid
    g0, g1 = gbuf
    m0, m1 = mbuf
    av0, av1 = avb
    ad0, ad1 = adb

    pltpu.sync_copy(si_hbm.at[pl.ds(pl.multiple_of(tg * _EPT, 8), _EPT)],
                    sbuf)
    pltpu.sync_copy(di_hbm.at[tg], dbuf)
    pltpu.sync_copy(b_hbm, bv)

    def _zrow(r, carry):
        for c8 in range(8):
            g0[r, pl.ds(c8 * 16, 16)] = jnp.zeros((16,), jnp.float32)
        return carry

    lax.fori_loop(0, _K, _zrow, 0)
    for g in range(_K // 16 + 1):
        m0[pl.ds(g * 16, 16)] = jnp.zeros((16,), jnp.float32)

    for k in range(8):
        start = pl.multiple_of(sid * 8 * _K + k * _K, _K)

        @pl.when(start < _N)
        def _():
            pltpu.sync_copy(g0, acc_s.at[pl.ds(start, _K)])
            pltpu.sync_copy(m0.at[pl.ds(16, _K)], den_s.at[pl.ds(start, _K)])

    plsc.subcore_barrier()

    gs0, gs1, as0, as1, ds0, ds1, rs0, rs1, ms0, ms1 = sems
    P0 = (g0, av0, ad0, m0, gs0, as0, ds0, rs0, ms0)
    P1 = (g1, av1, ad1, m1, gs1, as1, ds1, rs1, ms1)

    def srow(j1):
        return sbuf.at[pl.ds(pl.multiple_of(j1 * _K, 8), _K)]

    def fire_in(j1, P):
        gb, vb, db, gsem, asem, dsem = P[0], P[1], P[2], P[4], P[5], P[6]
        pltpu.async_copy(h_hbm.at[srow(j1)], gb, gsem)
        pltpu.async_copy(as_hbm.at[srow(j1)], vb, asem)
        pltpu.async_copy(ad_hbm.at[dbuf.at[j1]], db, dsem)

    fire_in(0, P0)

    def run_chunk(j, P, Q):
        gb, vb, db, mb, gsem, asem, dsem, rsem, msem = P
        qgb, qrsem = Q[0], Q[7]
        pltpu.make_async_copy(h_hbm.at[srow(j)], gb, gsem).wait()
        pltpu.make_async_copy(as_hbm.at[srow(j)], vb, asem).wait()
        pltpu.make_async_copy(ad_hbm.at[dbuf.at[j]], db, dsem).wait()

        @pl.when(j + 1 < _NCH)
        def _():
            @pl.when(j >= 1)
            def _():
                pltpu.make_async_copy(qgb, acc_s.at[dbuf.at[j]],
                                      qrsem).wait()

            fire_in(j + 1, Q)

        @pl.when(j >= 2)
        def _():
            pltpu.make_async_copy(mb.at[pl.ds(16, _K)],
                                  den_s.at[dbuf.at[j]], msem).wait()

        bb = bv[...]
        for g in range(_K // 16):
            e = vb[pl.ds(g * 16, 16)] + db[pl.ds(g * 16, 16)]
            e = jnp.where(e > 0, e, e * jnp.float32(0.2))
            mb[pl.ds(16 + g * 16, 16)] = jnp.exp(e - bb)
        for k in range(_K):
            mk = plsc.load_gather(mb, [jnp.full((16,), 16 + k, jnp.int32)])
            for r in range(8):
                gb[k, pl.ds(r * 16, 16)] = gb[k, pl.ds(r * 16, 16)] * mk
        pltpu.async_copy(gb, acc_s.at[dbuf.at[j]], rsem, add=True)
        pltpu.async_copy(mb.at[pl.ds(16, _K)], den_s.at[dbuf.at[j]], msem,
                         add=True)

    def _loop(j, carry):
        @pl.when(j % 2 == 0)
        def _():
            run_chunk(j, P0, P1)

        @pl.when(j % 2 == 1)
        def _():
            run_chunk(j, P1, P0)

        return carry

    lax.fori_loop(0, _NCH, _loop, 0)
    pltpu.make_async_copy(g0, acc_s.at[dbuf.at[0]], rs0).wait()
    pltpu.make_async_copy(g1, acc_s.at[dbuf.at[0]], rs1).wait()
    pltpu.make_async_copy(m0.at[pl.ds(16, _K)], den_s.at[dbuf.at[0]],
                          ms0).wait()
    pltpu.make_async_copy(m1.at[pl.ds(16, _K)], den_s.at[dbuf.at[0]],
                          ms1).wait()
    plsc.subcore_barrier()

    for k in range(8):
        start = pl.multiple_of(sid * 8 * _K + k * _K, _K)

        @pl.when(start < _N)
        def _():
            pltpu.sync_copy(acc_s.at[pl.ds(start, _K)],
                            acc_out.at[cid, pl.ds(start, _K)])

    @pl.when(sid == 0)
    def _():
        pltpu.sync_copy(den_s, den_out.at[cid, 0])


def _gat_sc(h, si, di, a_s, a_d, bvec):
    mesh = plsc.VectorSubcoreMesh(core_axis_name="c", subcore_axis_name="s")
    f = pl.kernel(
        _gat_sc_body,
        out_type=[
            jax.ShapeDtypeStruct((2, _N, _C), jnp.float32),
            jax.ShapeDtypeStruct((2, 1, _N), jnp.float32),
        ],
        mesh=mesh,
        compiler_params=pltpu.CompilerParams(needs_layout_passes=False),
        scratch_types=[
            pltpu.VMEM((_EPT,), jnp.int32),
            pltpu.VMEM((_NCH, _K), jnp.int32),
            (pltpu.VMEM((_K, _C), jnp.float32),
             pltpu.VMEM((_K, _C), jnp.float32)),
            (pltpu.VMEM((16 + _K,), jnp.float32),
             pltpu.VMEM((16 + _K,), jnp.float32)),
            (pltpu.VMEM((_K,), jnp.float32),
             pltpu.VMEM((_K,), jnp.float32)),
            (pltpu.VMEM((_K,), jnp.float32),
             pltpu.VMEM((_K,), jnp.float32)),
            pltpu.VMEM((16,), jnp.float32),
            pltpu.VMEM_SHARED((_N, _C), jnp.float32),
            pltpu.VMEM_SHARED((_N,), jnp.float32),
        ] + [pltpu.SemaphoreType.DMA] * 10,
    )
    return f(h, si, di, a_s, a_d, bvec)


def _epi_body(acc_ref, den_ref, bias_ref, o_ref):
    rows = acc_ref[0] + acc_ref[1]
    den = den_ref[0] + den_ref[1]
    o_ref[...] = jnp.tanh(rows / (den + jnp.float32(1e-16)) + bias_ref[...])


def _epi(acc, den, bias2):
    grid = _N // _BN
    return pl.pallas_call(
        _epi_body,
        grid=(grid,),
        in_specs=[
            pl.BlockSpec((2, _BN, _C), lambda i: (0, i, 0)),
            pl.BlockSpec((2, _BN, 1), lambda i: (0, i, 0)),
            pl.BlockSpec((1, _C), lambda i: (0, 0)),
        ],
        out_specs=pl.BlockSpec((_BN, _C), lambda i: (i, 0)),
        out_shape=jax.ShapeDtypeStruct((_N, _C), jnp.float32),
    )(acc, den, bias2)


def kernel(x, edge_index, W_src, W_dst, att_src, att_dst, bias):
    n, d = x.shape
    c = W_src.shape[1]
    h, a_s, a_d, bmax = _proj(x, W_src, W_dst,
                              att_src.reshape(1, c), att_dst.reshape(1, c))
    si = edge_index[0]
    di = edge_index[1].reshape(_NT, _NCH, _K)
    acc, den = _gat_sc(h, si, di, a_s.reshape(n), a_d.reshape(n),
                       bmax.reshape(16))
    return _epi(acc, den.reshape(2, n, 1), bias.reshape(1, c))


# speedup vs baseline: 1.5447x; 1.0030x over previous
import jax
import jax.numpy as jnp
from jax import lax
from jax.experimental import pallas as pl
from jax.experimental.pallas import tpu as pltpu
from jax.experimental.pallas import tpu_sc as plsc

_N = 10000
_E = 320000
_C = 128
_NT = 32
_EPT = _E // _NT
_K = 80
_NCH = _EPT // _K
_BN = 2000


def _proj_body(x_ref, ws_ref, wd_ref, atts_ref, attd_ref,
               h_ref, as_ref, ad_ref, b_ref, ms_ref, md_ref):
    i = pl.program_id(0)
    ng = pl.num_programs(0)
    x = x_ref[...]
    h = jnp.dot(x, ws_ref[...], preferred_element_type=jnp.float32)
    h_ref[...] = h
    a_s = jnp.sum(h * atts_ref[...], axis=1, keepdims=True)
    as_ref[...] = a_s
    hd = jnp.dot(x, wd_ref[...], preferred_element_type=jnp.float32)
    a_d = jnp.sum(hd * attd_ref[...], axis=1, keepdims=True)
    ad_ref[...] = a_d

    @pl.when(i == 0)
    def _():
        ms_ref[0] = jnp.float32(-jnp.inf)
        md_ref[0] = jnp.float32(-jnp.inf)

    ms_ref[0] = jnp.maximum(ms_ref[0], jnp.max(a_s))
    md_ref[0] = jnp.maximum(md_ref[0], jnp.max(a_d))

    @pl.when(i == ng - 1)
    def _():
        b = jnp.maximum(ms_ref[0] + md_ref[0], 0.0)
        b_ref[...] = jnp.full((1, 16), b, jnp.float32)


def _proj(x, w_s, w_d, atts, attd):
    grid = _N // _BN
    return pl.pallas_call(
        _proj_body,
        grid=(grid,),
        in_specs=[
            pl.BlockSpec((_BN, _C), lambda i: (i, 0)),
            pl.BlockSpec((_C, _C), lambda i: (0, 0)),
            pl.BlockSpec((_C, _C), lambda i: (0, 0)),
            pl.BlockSpec((1, _C), lambda i: (0, 0)),
            pl.BlockSpec((1, _C), lambda i: (0, 0)),
        ],
        out_specs=[
            pl.BlockSpec((_BN, _C), lambda i: (i, 0)),
            pl.BlockSpec((_BN, 1), lambda i: (i, 0)),
            pl.BlockSpec((_BN, 1), lambda i: (i, 0)),
            pl.BlockSpec((1, 16), lambda i: (0, 0)),
        ],
        out_shape=[
            jax.ShapeDtypeStruct((_N, _C), jnp.float32),
            jax.ShapeDtypeStruct((_N, 1), jnp.float32),
            jax.ShapeDtypeStruct((_N, 1), jnp.float32),
            jax.ShapeDtypeStruct((1, 16), jnp.float32),
        ],
        scratch_shapes=[
            pltpu.SMEM((1,), jnp.float32),
            pltpu.SMEM((1,), jnp.float32),
        ],
    )(x, w_s, w_d, atts, attd)


def _gat_sc_body(h_hbm, si_hbm, di_hbm, as_hbm, ad_hbm, b_hbm,
                 acc_out, den_out,
                 sbuf, dbuf, gbuf, mbuf, avb, adb, bv,
                 acc_s, den_s, *sems):
    cid = lax.axis_index("c")
    sid = lax.axis_index("s")
    tg = cid * 16 + sid
    g0, g1 = gbuf
    m0, m1 = mbuf
    av0, av1 = avb
    ad0, ad1 = adb

    pltpu.sync_copy(si_hbm.at[pl.ds(pl.multiple_of(tg * _EPT, 8), _EPT)],
                    sbuf)
    pltpu.sync_copy(di_hbm.at[tg], dbuf)
    pltpu.sync_copy(b_hbm, bv)

    def _zrow(r, carry):
        for c8 in range(8):
            g0[r, pl.ds(c8 * 16, 16)] = jnp.zeros((16,), jnp.float32)
        return carry

    lax.fori_loop(0, _K, _zrow, 0)
    for g in range(_K // 16 + 1):
        m0[pl.ds(g * 16, 16)] = jnp.zeros((16,), jnp.float32)

    for k in range(8):
        start = pl.multiple_of(sid * 8 * _K + k * _K, _K)

        @pl.when(start < _N)
        def _():
            pltpu.sync_copy(g0, acc_s.at[pl.ds(start, _K)])
            pltpu.sync_copy(m0.at[pl.ds(16, _K)], den_s.at[pl.ds(start, _K)])

    plsc.subcore_barrier()

    gs0, gs1, as0, as1, ds0, ds1, rs0, rs1, ms0, ms1 = sems
    P0 = (g0, av0, ad0, m0, gs0, as0, ds0, rs0, ms0)
    P1 = (g1, av1, ad1, m1, gs1, as1, ds1, rs1, ms1)

    def srow(j1):
        return sbuf.at[pl.ds(pl.multiple_of(j1 * _K, 8), _K)]

    def fire_in(j1, P):
        gb, vb, db, gsem, asem, dsem = P[0], P[1], P[2], P[4], P[5], P[6]
        pltpu.async_copy(h_hbm.at[srow(j1)], gb, gsem)
        pltpu.async_copy(as_hbm.at[srow(j1)], vb, asem)
        pltpu.async_copy(ad_hbm.at[dbuf.at[j1]], db, dsem)

    fire_in(0, P0)

    def run_chunk(j, P, Q):
        gb, vb, db, mb, gsem, asem, dsem, rsem, msem = P
        qgb, qrsem = Q[0], Q[7]
        pltpu.make_async_copy(h_hbm.at[srow(j)], gb, gsem).wait()
        pltpu.make_async_copy(as_hbm.at[srow(j)], vb, asem).wait()
        pltpu.make_async_copy(ad_hbm.at[dbuf.at[j]], db, dsem).wait()

        @pl.when(j + 1 < _NCH)
        def _():
            @pl.when(j >= 1)
            def _():
                pltpu.make_async_copy(qgb, acc_s.at[dbuf.at[j]],
                                      qrsem).wait()

            fire_in(j + 1, Q)

        @pl.when(j >= 2)
        def _():
            pltpu.make_async_copy(mb.at[pl.ds(16, _K)],
                                  den_s.at[dbuf.at[j]], msem).wait()

        bb = bv[...]
        for g in range(_K // 16):
            e = vb[pl.ds(g * 16, 16)] + db[pl.ds(g * 16, 16)]
            e = jnp.where(e > 0, e, e * jnp.float32(0.2))
            mb[pl.ds(16 + g * 16, 16)] = jnp.exp(e - bb)
        for k in range(_K):
            mk = plsc.load_gather(mb, [jnp.full((16,), 16 + k, jnp.int32)])
            for r in range(8):
                gb[k, pl.ds(r * 16, 16)] = gb[k, pl.ds(r * 16, 16)] * mk
        pltpu.async_copy(gb, acc_s.at[dbuf.at[j]], rsem, add=True)
        pltpu.async_copy(mb.at[pl.ds(16, _K)], den_s.at[dbuf.at[j]], msem,
                         add=True)

    def _loop(j, carry):
        @pl.when(j % 2 == 0)
        def _():
            run_chunk(j, P0, P1)

        @pl.when(j % 2 == 1)
        def _():
            run_chunk(j, P1, P0)

        return carry

    lax.fori_loop(0, _NCH, _loop, 0)
    pltpu.make_async_copy(g0, acc_s.at[dbuf.at[0]], rs0).wait()
    pltpu.make_async_copy(g1, acc_s.at[dbuf.at[0]], rs1).wait()
    pltpu.make_async_copy(m0.at[pl.ds(16, _K)], den_s.at[dbuf.at[0]],
                          ms0).wait()
    pltpu.make_async_copy(m1.at[pl.ds(16, _K)], den_s.at[dbuf.at[0]],
                          ms1).wait()
    plsc.subcore_barrier()

    for k in range(8):
        start = pl.multiple_of(sid * 8 * _K + k * _K, _K)

        @pl.when(start < _N)
        def _():
            pltpu.sync_copy(acc_s.at[pl.ds(start, _K)],
                            acc_out.at[cid, pl.ds(start, _K)])

    @pl.when(sid == 0)
    def _():
        pltpu.sync_copy(den_s, den_out.at[cid, 0])


def _gat_sc(h, si, di, a_s, a_d, bvec):
    mesh = plsc.VectorSubcoreMesh(core_axis_name="c", subcore_axis_name="s")
    f = pl.kernel(
        _gat_sc_body,
        out_type=[
            jax.ShapeDtypeStruct((2, _N, _C), jnp.float32),
            jax.ShapeDtypeStruct((2, 1, _N), jnp.float32),
        ],
        mesh=mesh,
        compiler_params=pltpu.CompilerParams(needs_layout_passes=False),
        scratch_types=[
            pltpu.VMEM((_EPT,), jnp.int32),
            pltpu.VMEM((_NCH, _K), jnp.int32),
            (pltpu.VMEM((_K, _C), jnp.float32),
             pltpu.VMEM((_K, _C), jnp.float32)),
            (pltpu.VMEM((16 + _K,), jnp.float32),
             pltpu.VMEM((16 + _K,), jnp.float32)),
            (pltpu.VMEM((_K,), jnp.float32),
             pltpu.VMEM((_K,), jnp.float32)),
            (pltpu.VMEM((_K,), jnp.float32),
             pltpu.VMEM((_K,), jnp.float32)),
            pltpu.VMEM((16,), jnp.float32),
            pltpu.VMEM_SHARED((_N, _C), jnp.float32),
            pltpu.VMEM_SHARED((_N,), jnp.float32),
        ] + [pltpu.SemaphoreType.DMA] * 10,
    )
    return f(h, si, di, a_s, a_d, bvec)


def _epi_body(acc_ref, den_ref, bias_ref, o_ref):
    rows = acc_ref[0] + acc_ref[1]
    den = den_ref[0] + den_ref[1]
    o_ref[...] = jnp.tanh(rows / (den + jnp.float32(1e-16)) + bias_ref[...])


def _epi(acc, den, bias2):
    grid = _N // _BN
    return pl.pallas_call(
        _epi_body,
        grid=(grid,),
        in_specs=[
            pl.BlockSpec((2, _BN, _C), lambda i: (0, i, 0)),
            pl.BlockSpec((2, _BN, 1), lambda i: (0, i, 0)),
            pl.BlockSpec((1, _C), lambda i: (0, 0)),
        ],
        out_specs=pl.BlockSpec((_BN, _C), lambda i: (i, 0)),
        out_shape=jax.ShapeDtypeStruct((_N, _C), jnp.float32),
    )(acc, den, bias2)


def kernel(x, edge_index, W_src, W_dst, att_src, att_dst, bias):
    n, d = x.shape
    c = W_src.shape[1]
    h, a_s, a_d, bmax = _proj(x, W_src, W_dst,
                              att_src.reshape(1, c), att_dst.reshape(1, c))
    si = edge_index[0]
    di = edge_index[1].reshape(_NT, _NCH, _K)
    acc, den = _gat_sc(h, si, di, a_s.reshape(n), a_d.reshape(n),
                       bmax.reshape(16))
    return _epi(acc, den.reshape(2, n, 1), bias.reshape(1, c))
